# Initial kernel scaffold; baseline (speedup 1.0000x reference)
import jax
import jax.numpy as jnp
from jax.experimental import pallas as pl


def kernel(seq, type_emb, node_a_emb, node_b_emb, vp_w, vp_b, proj_w, proj_b):
    raise NotImplementedError("write your pallas kernel here")


# speedup vs baseline: 4.1388x; 4.1388x over previous
import functools

import jax
import jax.numpy as jnp
from jax import lax
from jax.experimental import pallas as pl
from jax.experimental.pallas import tpu as pltpu
from jax.experimental.pallas import tpu_sc as plsc

N_TOKENS = 100000
D = 128
D4 = 32
B, L = 4096, 200
N = B * L

NC, NS, LANES = 2, 16, 16
NW = NC * NS
NPW = N // NW
K = 128
CHUNKS = NPW // K


_ROWS = 10000


def _tt_body(tbl_ref, w_ref, out_ref):
    out_ref[...] = jnp.dot(tbl_ref[...], w_ref[0],
                           preferred_element_type=jnp.float32)


def _make_tt(big_table, w3):
    grid = (3 * N_TOKENS) // _ROWS
    return pl.pallas_call(
        _tt_body,
        grid=(grid,),
        in_specs=[
            pl.BlockSpec((_ROWS, D4), lambda i: (i, 0)),
            pl.BlockSpec((1, D4, D), lambda i: ((i * _ROWS) // N_TOKENS, 0, 0)),
        ],
        out_specs=pl.BlockSpec((_ROWS, D), lambda i: (i, 0)),
        out_shape=jax.ShapeDtypeStruct((3 * N_TOKENS, D), jnp.float32),
    )(big_table, w3)


def _uc_body(p_ref, w_ref, pb_ref, out_ref):
    out_ref[...] = jnp.dot(p_ref[...], w_ref[...],
                           preferred_element_type=jnp.float32) + pb_ref[...]


def _make_uc(p8, w3v, pb8):
    return pl.pallas_call(
        _uc_body,
        out_shape=jax.ShapeDtypeStruct((8, D), jnp.float32),
    )(p8, w3v, pb8)


def _sc_body(seqt_hbm, tt_hbm, uc_hbm, out_hbm,
             tf_v, af_v, bf_v, vv_v, ti_v, ai_v, bi_v,
             rt_v, ra_v, rb_v, uc_v, sem):
    wid = lax.axis_index("s") * NC + lax.axis_index("c")
    w_base = wid * NPW

    pltpu.sync_copy(uc_hbm, uc_v)
    u_rows = [uc_v[0, pl.ds(16 * j, 16)] for j in range(D // 16)]
    c_rows = [uc_v[1, pl.ds(16 * j, 16)] for j in range(D // 16)]

    def chunk_body(ci, carry):
        base = w_base + ci * K
        pltpu.sync_copy(seqt_hbm.at[0, pl.ds(base, K)], tf_v)
        pltpu.sync_copy(seqt_hbm.at[1, pl.ds(base, K)], af_v)
        pltpu.sync_copy(seqt_hbm.at[2, pl.ds(base, K)], bf_v)
        pltpu.sync_copy(seqt_hbm.at[3, pl.ds(base, K)], vv_v)

        for g in range(K // LANES):
            s = pl.ds(g * LANES, LANES)
            ti_v[s] = jnp.clip(tf_v[s].astype(jnp.int32), 0, N_TOKENS - 1)
            ai_v[s] = jnp.clip(af_v[s].astype(jnp.int32), 0, N_TOKENS - 1) \
                + N_TOKENS
            bi_v[s] = jnp.clip(bf_v[s].astype(jnp.int32), 0, N_TOKENS - 1) \
                + 2 * N_TOKENS

        cp1 = pltpu.async_copy(tt_hbm.at[ti_v], rt_v, sem)
        cp2 = pltpu.async_copy(tt_hbm.at[ai_v], ra_v, sem)
        cp3 = pltpu.async_copy(tt_hbm.at[bi_v], rb_v, sem)
        cp1.wait()
        cp2.wait()
        cp3.wait()

        def grp_body(g, carry2):
            vblk = vv_v[pl.ds(g * LANES, LANES)]
            for t in range(LANES):
                k = g * LANES + t
                vk = jnp.broadcast_to(vblk[t], (LANES,))
                for j in range(D // 16):
                    s = pl.ds(16 * j, 16)
                    rt_v[k, s] = (rt_v[k, s] + ra_v[k, s] + rb_v[k, s]
                                  + vk * u_rows[j] + c_rows[j])
            return carry2

        lax.fori_loop(0, K // LANES, grp_body, 0)
        pltpu.sync_copy(rt_v, out_hbm.at[pl.ds(base, K)])
        return carry

    lax.fori_loop(0, CHUNKS, chunk_body, 0)


@functools.cache
def _sc_gather_fn():
    return pl.kernel(
        _sc_body,
        out_type=jax.ShapeDtypeStruct((N, D), jnp.float32),
        mesh=plsc.VectorSubcoreMesh(core_axis_name="c", subcore_axis_name="s",
                                    num_cores=NC, num_subcores=NS),
        scratch_types=[
        pltpu.VMEM((K,), jnp.float32),
        pltpu.VMEM((K,), jnp.float32),
        pltpu.VMEM((K,), jnp.float32),
        pltpu.VMEM((K,), jnp.float32),
        pltpu.VMEM((K,), jnp.int32),
        pltpu.VMEM((K,), jnp.int32),
        pltpu.VMEM((K,), jnp.int32),
        pltpu.VMEM((K, D), jnp.float32),
        pltpu.VMEM((K, D), jnp.float32),
        pltpu.VMEM((K, D), jnp.float32),
            pltpu.VMEM((8, D), jnp.float32),
            pltpu.SemaphoreType.DMA,
        ],
    )


def kernel(seq, type_emb, node_a_emb, node_b_emb, vp_w, vp_b, proj_w, proj_b):
    big_table = jnp.concatenate([type_emb, node_a_emb, node_b_emb], axis=0)
    w3 = proj_w[: 3 * D4].reshape(3, D4, D)
    tt = _make_tt(big_table, w3)

    p8 = jnp.zeros((8, D4), jnp.float32).at[0].set(vp_w[0]).at[1].set(vp_b)
    pb8 = jnp.zeros((8, D), jnp.float32).at[1].set(proj_b)
    uc = _make_uc(p8, proj_w[3 * D4:], pb8)

    seqt = seq.reshape(N, 4).T
    out = _sc_gather_fn()(seqt, tt, uc)
    return out.reshape(B, L, D)


# speedup vs baseline: 6.6050x; 1.5958x over previous
import functools

import jax
import jax.numpy as jnp
from jax import lax
from jax.experimental import pallas as pl
from jax.experimental.pallas import tpu as pltpu
from jax.experimental.pallas import tpu_sc as plsc

N_TOKENS = 100000
D = 128
D4 = 32
B, L = 4096, 200
N = B * L

NC, NS, LANES = 2, 16, 16
NW = NC * NS
NPW = N // NW
K = 80
CHUNKS = NPW // K


_ROWS = 10000


def _tt_body(tbl_ref, w_ref, out_ref):
    out_ref[...] = jnp.dot(tbl_ref[...], w_ref[0],
                           preferred_element_type=jnp.float32)


def _make_tt(big_table, w3):
    grid = (3 * N_TOKENS) // _ROWS
    return pl.pallas_call(
        _tt_body,
        grid=(grid,),
        in_specs=[
            pl.BlockSpec((_ROWS, D4), lambda i: (i, 0)),
            pl.BlockSpec((1, D4, D), lambda i: ((i * _ROWS) // N_TOKENS, 0, 0)),
        ],
        out_specs=pl.BlockSpec((_ROWS, D), lambda i: (i, 0)),
        out_shape=jax.ShapeDtypeStruct((3 * N_TOKENS, D), jnp.float32),
    )(big_table, w3)


def _uc_body(p_ref, w_ref, pb_ref, out_ref):
    out_ref[...] = jnp.dot(p_ref[...], w_ref[...],
                           preferred_element_type=jnp.float32) + pb_ref[...]


def _make_uc(p8, w3v, pb8):
    return pl.pallas_call(
        _uc_body,
        out_shape=jax.ShapeDtypeStruct((8, D), jnp.float32),
    )(p8, w3v, pb8)


def _sc_body(seqb_hbm, tt_hbm, uc_hbm, out_hbm,
             seq0, seq1, ti0, ti1, ai0, ai1, bi0, bi1,
             rt0, rt1, ra0, ra1, rb0, rb1, st0, st1, uc_v,
             gs0, gs1, ss0, ss1):
    seqv = [seq0, seq1]
    tiv = [ti0, ti1]
    aiv = [ai0, ai1]
    biv = [bi0, bi1]
    rtv = [rt0, rt1]
    rav = [ra0, ra1]
    rbv = [rb0, rb1]
    stv = [st0, st1]
    gsem = [gs0, gs1]
    ssem = [ss0, ss1]

    wid = lax.axis_index("s") * NC + lax.axis_index("c")
    c_base = wid * CHUNKS

    pltpu.sync_copy(uc_hbm, uc_v)
    u_rows = [uc_v[0, pl.ds(16 * j, 16)] for j in range(D // 16)]
    c_rows = [uc_v[1, pl.ds(16 * j, 16)] for j in range(D // 16)]

    def load_and_fire(ci, b):
        pltpu.sync_copy(seqb_hbm.at[c_base + ci], seqv[b])
        for g in range(K // LANES):
            s = pl.ds(g * LANES, LANES)
            tiv[b][s] = jnp.clip(seqv[b][0, s].astype(jnp.int32),
                                 0, N_TOKENS - 1)
            aiv[b][s] = jnp.clip(seqv[b][1, s].astype(jnp.int32),
                                 0, N_TOKENS - 1) + N_TOKENS
            biv[b][s] = jnp.clip(seqv[b][2, s].astype(jnp.int32),
                                 0, N_TOKENS - 1) + 2 * N_TOKENS
        pltpu.async_copy(tt_hbm.at[tiv[b]], rtv[b], gsem[b])
        pltpu.async_copy(tt_hbm.at[aiv[b]], rav[b], gsem[b])
        pltpu.async_copy(tt_hbm.at[biv[b]], rbv[b], gsem[b])

    def wait_gathers(b):
        pltpu.make_async_copy(tt_hbm.at[tiv[b]], rtv[b], gsem[b]).wait()
        pltpu.make_async_copy(tt_hbm.at[aiv[b]], rav[b], gsem[b]).wait()
        pltpu.make_async_copy(tt_hbm.at[biv[b]], rbv[b], gsem[b]).wait()

    def compute(b):
        def grp_body(g, carry2):
            vblk = seqv[b][3, pl.ds(g * LANES, LANES)]
            for t in range(LANES):
                k = g * LANES + t
                vk = jnp.broadcast_to(vblk[t], (LANES,))
                for j in range(D // 16):
                    s = pl.ds(16 * j, 16)
                    stv[b][k, s] = (rtv[b][k, s] + rav[b][k, s]
                                    + rbv[b][k, s] + vk * u_rows[j]
                                    + c_rows[j])
            return carry2

        lax.fori_loop(0, K // LANES, grp_body, 0)

    def out_slice(ci):
        return out_hbm.at[pl.ds((c_base + ci) * K, K)]

    def fire_store(ci, b):
        pltpu.async_copy(stv[b], out_slice(ci), ssem[b])

    def wait_store(ci, b):
        pltpu.make_async_copy(stv[b], out_slice(ci), ssem[b]).wait()

    load_and_fire(0, 0)

    def pair_body(p, carry):
        ci = 2 * p

        @pl.when(ci >= 2)
        def _():
            wait_store(ci - 2, 0)

        load_and_fire(ci + 1, 1)
        wait_gathers(0)
        compute(0)
        fire_store(ci, 0)

        @pl.when(ci >= 1)
        def _():
            wait_store(ci - 1, 1)

        @pl.when(ci + 2 < CHUNKS)
        def _():
            load_and_fire(ci + 2, 0)

        wait_gathers(1)
        compute(1)
        fire_store(ci + 1, 1)
        return carry

    lax.fori_loop(0, CHUNKS // 2, pair_body, 0)
    wait_store(CHUNKS - 2, 0)
    wait_store(CHUNKS - 1, 1)


@functools.cache
def _sc_gather_fn():
    return pl.kernel(
        _sc_body,
        out_type=jax.ShapeDtypeStruct((N, D), jnp.float32),
        mesh=plsc.VectorSubcoreMesh(core_axis_name="c", subcore_axis_name="s",
                                    num_cores=NC, num_subcores=NS),
        scratch_types=[
            pltpu.VMEM((4, K), jnp.float32),
            pltpu.VMEM((4, K), jnp.float32),
            pltpu.VMEM((K,), jnp.int32),
            pltpu.VMEM((K,), jnp.int32),
            pltpu.VMEM((K,), jnp.int32),
            pltpu.VMEM((K,), jnp.int32),
            pltpu.VMEM((K,), jnp.int32),
            pltpu.VMEM((K,), jnp.int32),
            pltpu.VMEM((K, D), jnp.float32),
            pltpu.VMEM((K, D), jnp.float32),
            pltpu.VMEM((K, D), jnp.float32),
            pltpu.VMEM((K, D), jnp.float32),
            pltpu.VMEM((K, D), jnp.float32),
            pltpu.VMEM((K, D), jnp.float32),
            pltpu.VMEM((K, D), jnp.float32),
            pltpu.VMEM((K, D), jnp.float32),
            pltpu.VMEM((8, D), jnp.float32),
            pltpu.SemaphoreType.DMA,
            pltpu.SemaphoreType.DMA,
            pltpu.SemaphoreType.DMA,
            pltpu.SemaphoreType.DMA,
        ],
    )


def kernel(seq, type_emb, node_a_emb, node_b_emb, vp_w, vp_b, proj_w, proj_b):
    big_table = jnp.concatenate([type_emb, node_a_emb, node_b_emb], axis=0)
    w3 = proj_w[: 3 * D4].reshape(3, D4, D)
    tt = _make_tt(big_table, w3)

    p8 = jnp.zeros((8, D4), jnp.float32).at[0].set(vp_w[0]).at[1].set(vp_b)
    pb8 = jnp.zeros((8, D), jnp.float32).at[1].set(proj_b)
    uc = _make_uc(p8, proj_w[3 * D4:], pb8)

    seqb = seq.reshape(N // K, K, 4).transpose(0, 2, 1)
    out = _sc_gather_fn()(seqb, tt, uc)
    return out.reshape(B, L, D)
